# baseline (speedup 1.0000x reference)
import functools

import jax
import jax.numpy as jnp
from jax import lax
from jax.experimental import pallas as pl
from jax.experimental.pallas import tpu as pltpu
from jax.experimental.pallas import tpu_sc as plsc

LANES = 16
KPAD = 32


def _make_sc_dots(B, K, E):
    info = plsc.get_sparse_core_info()
    NW = info.num_cores * info.num_subcores
    rows_per_w = B // NW
    C = 32
    n_chunks = rows_per_w // C
    E2 = 2 * E
    EV = E // LANES
    IDX_BLK = 128

    mesh = plsc.VectorSubcoreMesh(core_axis_name="c", subcore_axis_name="s")

    @functools.partial(
        pl.kernel,
        out_type=[
            jax.ShapeDtypeStruct((B,), jnp.float32),
            jax.ShapeDtypeStruct((B * KPAD,), jnp.float32),
        ],
        mesh=mesh,
        compiler_params=pltpu.CompilerParams(needs_layout_passes=False,
                                             use_tc_tiling_on_sc=True),
        scratch_types=[
            pltpu.VMEM((C,), jnp.int32),
            pltpu.VMEM((C,), jnp.int32),
            pltpu.VMEM((C * K,), jnp.int32),
            pltpu.VMEM((C,), jnp.int32),
            pltpu.VMEM((C,), jnp.int32),
            pltpu.VMEM((C * K,), jnp.int32),
            pltpu.VMEM((C * KPAD,), jnp.int32),
            pltpu.VMEM((C, E2), jnp.float32),
            pltpu.VMEM((C, E2), jnp.float32),
            pltpu.VMEM((C * K, E2), jnp.float32),
            pltpu.VMEM((C,), jnp.float32),
            pltpu.VMEM((C * KPAD,), jnp.float32),
            pltpu.SemaphoreType.DMA,
        ],
    )
    def sc_dots(center_hbm, context_hbm, negflat_hbm, inemb_hbm, outemb_hbm,
                pos_hbm, negdot_hbm,
                cidx, oidx, nidx, chalf, ohalf, nhalf, npar,
                crow, orow, nrow, posv, negv, sem):
        wid = lax.axis_index("s") * info.num_cores + lax.axis_index("c")
        wbase = wid * rows_per_w
        lane = lax.iota(jnp.int32, LANES)

        def halve(src, dst, n):
            for j in range(n // LANES):
                dst[pl.ds(j * LANES, LANES)] = (
                    lax.shift_right_logical(src[pl.ds(j * LANES, LANES)], 1))

        def dot_rows(a_ref, a_row, a_off, b_ref, b_row, b_off):
            acc = (a_ref[a_row, pl.ds(a_off, LANES)]
                   * b_ref[b_row, pl.ds(b_off, LANES)])
            for v in range(1, EV):
                acc = acc + (a_ref[a_row, pl.ds(a_off + v * LANES, LANES)]
                             * b_ref[b_row, pl.ds(b_off + v * LANES, LANES)])
            return jnp.sum(acc, axis=0)

        def chunk_body(g, _):
            base = wbase + g * C

            pltpu.sync_copy(center_hbm.at[pl.ds(base, C)], cidx)
            pltpu.sync_copy(context_hbm.at[pl.ds(base, C)], oidx)
            pltpu.sync_copy(negflat_hbm.at[pl.ds(base * K, C * K)], nidx)
            halve(cidx, chalf, C)
            halve(oidx, ohalf, C)
            halve(nidx, nhalf, C * K)
            for j in range(C * K // LANES):
                p = j * LANES + lane
                vals = (nidx[pl.ds(j * LANES, LANES)] & 1) * E
                dst = (p // K) * KPAD + (p % K)
                plsc.store_scatter(npar, [dst], vals)

            copies = [
                pltpu.async_copy(inemb_hbm.at[chalf], crow, sem),
                pltpu.async_copy(outemb_hbm.at[ohalf], orow, sem),
            ]
            for j in range(C * K // IDX_BLK):
                copies.append(pltpu.async_copy(
                    outemb_hbm.at[nhalf.at[pl.ds(j * IDX_BLK, IDX_BLK)]],
                    nrow.at[pl.ds(j * IDX_BLK, IDX_BLK)],
                    sem))
            for cp in copies:
                cp.wait()

            def grp_body(grp, _):
                r0 = grp * LANES
                cparv = (cidx[pl.ds(r0, LANES)] & 1) * E
                oparv = (oidx[pl.ds(r0, LANES)] & 1) * E
                pvec = jnp.zeros((LANES,), jnp.float32)
                for i in range(LANES):
                    r = r0 + i
                    c_off = cparv[i]
                    o_off = oparv[i]
                    pvec = jnp.where(
                        lane == i,
                        dot_rows(crow, r, c_off, orow, r, o_off), pvec)
                    npv0 = npar[pl.ds(r * KPAD, LANES)]
                    npv1 = npar[pl.ds(r * KPAD + LANES, LANES)]
                    nvec0 = jnp.zeros((LANES,), jnp.float32)
                    nvec1 = jnp.zeros((LANES,), jnp.float32)
                    for k in range(K):
                        n_off = npv0[k] if k < LANES else npv1[k - LANES]
                        s = dot_rows(crow, r, c_off, nrow, r * K + k, n_off)
                        if k < LANES:
                            nvec0 = jnp.where(lane == k, s, nvec0)
                        else:
                            nvec1 = jnp.where(lane == (k - LANES), s, nvec1)
                    negv[pl.ds(r * KPAD, LANES)] = nvec0
                    negv[pl.ds(r * KPAD + LANES, LANES)] = nvec1
                posv[pl.ds(r0, LANES)] = pvec
                return 0

            lax.fori_loop(0, C // LANES, grp_body, 0)

            pltpu.sync_copy(posv, pos_hbm.at[pl.ds(base, C)])
            pltpu.sync_copy(negv, negdot_hbm.at[pl.ds(base * KPAD, C * KPAD)])
            return 0

        lax.fori_loop(0, n_chunks, chunk_body, 0)

    return sc_dots


def _make_loss_body(B, K):
    def loss_body(pos_ref, neg_ref, out_ref):
        pos = pos_ref[...]
        neg = neg_ref[...]
        k_of_col = jax.lax.broadcasted_iota(jnp.int32, neg.shape, 1) % KPAD
        pos_term = -jnp.log(jax.nn.sigmoid(pos) + 1e-09)
        neg_term = jnp.where(k_of_col < K,
                             -jnp.log(jax.nn.sigmoid(-neg) + 1e-09), 0.0)
        out_ref[0, 0] = (jnp.sum(pos_term) + jnp.sum(neg_term)) / B
    return loss_body


def kernel(center, context, negatives, input_emb, output_emb):
    B, = center.shape
    K = negatives.shape[1]
    V, E = input_emb.shape

    sc_dots = _make_sc_dots(B, K, E)
    pos_dot, neg_dot = sc_dots(
        center.astype(jnp.int32),
        context.astype(jnp.int32),
        negatives.reshape(B * K).astype(jnp.int32),
        input_emb.reshape(V // 2, 2 * E),
        output_emb.reshape(V // 2, 2 * E),
    )

    loss = pl.pallas_call(
        _make_loss_body(B, K),
        out_shape=jax.ShapeDtypeStruct((1, 1), jnp.float32),
        in_specs=[
            pl.BlockSpec(memory_space=pltpu.VMEM),
            pl.BlockSpec(memory_space=pltpu.VMEM),
        ],
        out_specs=pl.BlockSpec(memory_space=pltpu.SMEM),
    )(pos_dot.reshape(B // 128, 128), neg_dot.reshape(B * KPAD // 128, 128))
    return loss.reshape(())


# speedup vs baseline: 1.6676x; 1.6676x over previous
import functools

import jax
import jax.numpy as jnp
from jax import lax
from jax.experimental import pallas as pl
from jax.experimental.pallas import tpu as pltpu
from jax.experimental.pallas import tpu_sc as plsc

LANES = 16
KPAD = 32
BI = 2048


def _make_relayout(V, E):
    NB = pl.cdiv(V, 2 * BI)
    H = NB * BI
    NBV = pl.cdiv(V, BI) - 1

    def body(up_ref, lo_ref, dst_ref):
        t_up = jnp.transpose(up_ref[...])
        t_lo = jnp.transpose(lo_ref[...])
        dst_ref[...] = jnp.concatenate([t_up, t_lo], axis=1)

    call = pl.pallas_call(
        body,
        grid=(NB,),
        in_specs=[
            pl.BlockSpec((E, BI), lambda i: (0, i)),
            pl.BlockSpec((E, BI), lambda i: (0, jnp.minimum(i + NB, NBV))),
        ],
        out_specs=pl.BlockSpec((BI, 2 * E), lambda i: (i, 0)),
        out_shape=jax.ShapeDtypeStruct((H, 2 * E), jnp.float32),
    )

    def relayout(table):
        tT = table.T
        return call(tT, tT).reshape(2 * H, E)

    return relayout, H


def _make_sc_dots(B, K, E, H):
    info = plsc.get_sparse_core_info()
    NW = info.num_cores * info.num_subcores
    rows_per_w = B // NW
    C = 64
    n_chunks = rows_per_w // C
    EV = E // LANES
    IDX_BLK = 128

    mesh = plsc.VectorSubcoreMesh(core_axis_name="c", subcore_axis_name="s")

    @functools.partial(
        pl.kernel,
        out_type=[
            jax.ShapeDtypeStruct((B,), jnp.float32),
            jax.ShapeDtypeStruct((B * KPAD,), jnp.float32),
        ],
        mesh=mesh,
        compiler_params=pltpu.CompilerParams(needs_layout_passes=False,
                                             use_tc_tiling_on_sc=False),
        scratch_types=[
            pltpu.VMEM((C,), jnp.int32),
            pltpu.VMEM((C,), jnp.int32),
            pltpu.VMEM((C * K,), jnp.int32),
            pltpu.VMEM((C, E), jnp.float32),
            pltpu.VMEM((C, E), jnp.float32),
            pltpu.VMEM((C * K, E), jnp.float32),
            pltpu.VMEM((C,), jnp.float32),
            pltpu.VMEM((C * KPAD,), jnp.float32),
            pltpu.SemaphoreType.DMA,
        ],
    )
    def sc_dots(center_hbm, context_hbm, negflat_hbm, inemb_hbm, outemb_hbm,
                pos_hbm, negdot_hbm,
                cidx, oidx, nidx, crow, orow, nrow, posv, negv, sem):
        wid = lax.axis_index("s") * info.num_cores + lax.axis_index("c")
        wbase = wid * rows_per_w
        lane = lax.iota(jnp.int32, LANES)

        def remap(ref, n):
            for j in range(n // LANES):
                v = ref[pl.ds(j * LANES, LANES)]
                sel = (v >= H).astype(jnp.int32)
                ref[pl.ds(j * LANES, LANES)] = v * 2 - sel * (2 * H - 1)

        def dot_rows(a_ref, a_row, b_ref, b_row):
            acc = a_ref[a_row, pl.ds(0, LANES)] * b_ref[b_row, pl.ds(0, LANES)]
            for v in range(1, EV):
                acc = acc + (a_ref[a_row, pl.ds(v * LANES, LANES)]
                             * b_ref[b_row, pl.ds(v * LANES, LANES)])
            return jnp.sum(acc, axis=0)

        def chunk_body(g, _):
            base = wbase + g * C

            pltpu.sync_copy(center_hbm.at[pl.ds(base, C)], cidx)
            pltpu.sync_copy(context_hbm.at[pl.ds(base, C)], oidx)
            pltpu.sync_copy(negflat_hbm.at[pl.ds(base * K, C * K)], nidx)
            remap(cidx, C)
            remap(oidx, C)
            remap(nidx, C * K)

            copies = [
                pltpu.async_copy(inemb_hbm.at[cidx], crow, sem),
                pltpu.async_copy(outemb_hbm.at[oidx], orow, sem),
            ]
            for j in range(C * K // IDX_BLK):
                copies.append(pltpu.async_copy(
                    outemb_hbm.at[nidx.at[pl.ds(j * IDX_BLK, IDX_BLK)]],
                    nrow.at[pl.ds(j * IDX_BLK, IDX_BLK)],
                    sem))
            for cp in copies:
                cp.wait()

            def grp_body(grp, _):
                r0 = grp * LANES
                pvec = jnp.zeros((LANES,), jnp.float32)
                for i in range(LANES):
                    r = r0 + i
                    pvec = jnp.where(lane == i, dot_rows(crow, r, orow, r),
                                     pvec)
                    nvec0 = jnp.zeros((LANES,), jnp.float32)
                    nvec1 = jnp.zeros((LANES,), jnp.float32)
                    for k in range(K):
                        s = dot_rows(crow, r, nrow, r * K + k)
                        if k < LANES:
                            nvec0 = jnp.where(lane == k, s, nvec0)
                        else:
                            nvec1 = jnp.where(lane == (k - LANES), s, nvec1)
                    negv[pl.ds(r * KPAD, LANES)] = nvec0
                    negv[pl.ds(r * KPAD + LANES, LANES)] = nvec1
                posv[pl.ds(r0, LANES)] = pvec
                return 0

            lax.fori_loop(0, C // LANES, grp_body, 0)

            pltpu.sync_copy(posv, pos_hbm.at[pl.ds(base, C)])
            pltpu.sync_copy(negv, negdot_hbm.at[pl.ds(base * KPAD, C * KPAD)])
            return 0

        lax.fori_loop(0, n_chunks, chunk_body, 0)

    return sc_dots


def _make_loss_body(B, K):
    def loss_body(pos_ref, neg_ref, out_ref):
        pos = pos_ref[...]
        neg = neg_ref[...]
        k_of_col = jax.lax.broadcasted_iota(jnp.int32, neg.shape, 1) % KPAD
        pos_term = -jnp.log(jax.nn.sigmoid(pos) + 1e-09)
        neg_term = jnp.where(k_of_col < K,
                             -jnp.log(jax.nn.sigmoid(-neg) + 1e-09), 0.0)
        out_ref[0, 0] = (jnp.sum(pos_term) + jnp.sum(neg_term)) / B
    return loss_body


def kernel(center, context, negatives, input_emb, output_emb):
    B, = center.shape
    K = negatives.shape[1]
    V, E = input_emb.shape

    relayout, H = _make_relayout(V, E)
    in_lin = relayout(input_emb)
    out_lin = relayout(output_emb)

    sc_dots = _make_sc_dots(B, K, E, H)
    pos_dot, neg_dot = sc_dots(
        center.astype(jnp.int32),
        context.astype(jnp.int32),
        negatives.reshape(B * K).astype(jnp.int32),
        in_lin,
        out_lin,
    )

    loss = pl.pallas_call(
        _make_loss_body(B, K),
        out_shape=jax.ShapeDtypeStruct((1, 1), jnp.float32),
        in_specs=[
            pl.BlockSpec(memory_space=pltpu.VMEM),
            pl.BlockSpec(memory_space=pltpu.VMEM),
        ],
        out_specs=pl.BlockSpec(memory_space=pltpu.SMEM),
    )(pos_dot.reshape(B // 128, 128), neg_dot.reshape(B * KPAD // 128, 128))
    return loss.reshape(())


# speedup vs baseline: 2.2109x; 1.3258x over previous
import functools

import jax
import jax.numpy as jnp
from jax import lax
from jax.experimental import pallas as pl
from jax.experimental.pallas import tpu as pltpu
from jax.experimental.pallas import tpu_sc as plsc

LANES = 16
KPAD = 32
BI = 8192


def _make_relayout(V, E):
    NB = pl.cdiv(V, 2 * BI)
    H = NB * BI
    NBV = pl.cdiv(V, BI) - 1

    def body(up_ref, lo_ref, dst_ref):
        t_up = jnp.transpose(up_ref[...])
        t_lo = jnp.transpose(lo_ref[...])
        dst_ref[...] = jnp.concatenate([t_up, t_lo], axis=1)

    call = pl.pallas_call(
        body,
        grid=(NB,),
        in_specs=[
            pl.BlockSpec((E, BI), lambda i: (0, i)),
            pl.BlockSpec((E, BI), lambda i: (0, jnp.minimum(i + NB, NBV))),
        ],
        out_specs=pl.BlockSpec((BI, 2 * E), lambda i: (i, 0)),
        out_shape=jax.ShapeDtypeStruct((H, 2 * E), jnp.float32),
    )

    def relayout(table):
        tT = table.T
        return call(tT, tT).reshape(2 * H, E)

    return relayout, H


def _make_sc_dots(B, K, E, H):
    info = plsc.get_sparse_core_info()
    NW = info.num_cores * info.num_subcores
    rows_per_w = B // NW
    C = 64
    n_chunks = rows_per_w // C
    EV = E // LANES
    IDX_BLK = 128

    mesh = plsc.VectorSubcoreMesh(core_axis_name="c", subcore_axis_name="s")

    @functools.partial(
        pl.kernel,
        out_type=[
            jax.ShapeDtypeStruct((B,), jnp.float32),
            jax.ShapeDtypeStruct((B * KPAD,), jnp.float32),
        ],
        mesh=mesh,
        compiler_params=pltpu.CompilerParams(needs_layout_passes=False,
                                             use_tc_tiling_on_sc=False),
        scratch_types=[
            pltpu.VMEM((C,), jnp.int32),
            pltpu.VMEM((C,), jnp.int32),
            pltpu.VMEM((C * K,), jnp.int32),
            pltpu.VMEM((C, E), jnp.float32),
            pltpu.VMEM((C, E), jnp.float32),
            pltpu.VMEM((C * K, E), jnp.float32),
            pltpu.VMEM((C,), jnp.float32),
            pltpu.VMEM((C * KPAD,), jnp.float32),
            pltpu.SemaphoreType.DMA,
        ],
    )
    def sc_dots(center_hbm, context_hbm, negflat_hbm, inemb_hbm, outemb_hbm,
                pos_hbm, negdot_hbm,
                cidx, oidx, nidx, crow, orow, nrow, posv, negv, sem):
        wid = lax.axis_index("s") * info.num_cores + lax.axis_index("c")
        wbase = wid * rows_per_w
        lane = lax.iota(jnp.int32, LANES)

        def remap(ref, n):
            for j in range(n // LANES):
                v = ref[pl.ds(j * LANES, LANES)]
                sel = (v >= H).astype(jnp.int32)
                ref[pl.ds(j * LANES, LANES)] = v * 2 - sel * (2 * H - 1)

        def dot_rows(a_ref, a_row, b_ref, b_row):
            acc = a_ref[a_row, pl.ds(0, LANES)] * b_ref[b_row, pl.ds(0, LANES)]
            for v in range(1, EV):
                acc = acc + (a_ref[a_row, pl.ds(v * LANES, LANES)]
                             * b_ref[b_row, pl.ds(v * LANES, LANES)])
            return jnp.sum(acc, axis=0)

        def chunk_body(g, _):
            base = wbase + g * C

            pltpu.sync_copy(center_hbm.at[pl.ds(base, C)], cidx)
            pltpu.sync_copy(context_hbm.at[pl.ds(base, C)], oidx)
            pltpu.sync_copy(negflat_hbm.at[pl.ds(base * K, C * K)], nidx)
            remap(cidx, C)
            remap(oidx, C)
            remap(nidx, C * K)

            copies = [
                pltpu.async_copy(inemb_hbm.at[cidx], crow, sem),
                pltpu.async_copy(outemb_hbm.at[oidx], orow, sem),
            ]
            for j in range(C * K // IDX_BLK):
                copies.append(pltpu.async_copy(
                    outemb_hbm.at[nidx.at[pl.ds(j * IDX_BLK, IDX_BLK)]],
                    nrow.at[pl.ds(j * IDX_BLK, IDX_BLK)],
                    sem))
            for cp in copies:
                cp.wait()

            def grp_body(grp, _):
                r0 = grp * LANES
                pvec = jnp.zeros((LANES,), jnp.float32)
                for i in range(LANES):
                    r = r0 + i
                    pvec = jnp.where(lane == i, dot_rows(crow, r, orow, r),
                                     pvec)
                    nvec0 = jnp.zeros((LANES,), jnp.float32)
                    nvec1 = jnp.zeros((LANES,), jnp.float32)
                    for k in range(K):
                        s = dot_rows(crow, r, nrow, r * K + k)
                        if k < LANES:
                            nvec0 = jnp.where(lane == k, s, nvec0)
                        else:
                            nvec1 = jnp.where(lane == (k - LANES), s, nvec1)
                    negv[pl.ds(r * KPAD, LANES)] = nvec0
                    negv[pl.ds(r * KPAD + LANES, LANES)] = nvec1
                posv[pl.ds(r0, LANES)] = pvec
                return 0

            lax.fori_loop(0, C // LANES, grp_body, 0)

            pltpu.sync_copy(posv, pos_hbm.at[pl.ds(base, C)])
            pltpu.sync_copy(negv, negdot_hbm.at[pl.ds(base * KPAD, C * KPAD)])
            return 0

        lax.fori_loop(0, n_chunks, chunk_body, 0)

    return sc_dots


def _make_loss_body(B, K):
    def loss_body(pos_ref, neg_ref, out_ref):
        pos = pos_ref[...]
        neg = neg_ref[...]
        k_of_col = jax.lax.broadcasted_iota(jnp.int32, neg.shape, 1) % KPAD
        pos_term = -jnp.log(jax.nn.sigmoid(pos) + 1e-09)
        neg_term = jnp.where(k_of_col < K,
                             -jnp.log(jax.nn.sigmoid(-neg) + 1e-09), 0.0)
        out_ref[0, 0] = (jnp.sum(pos_term) + jnp.sum(neg_term)) / B
    return loss_body


def kernel(center, context, negatives, input_emb, output_emb):
    B, = center.shape
    K = negatives.shape[1]
    V, E = input_emb.shape

    relayout, H = _make_relayout(V, E)
    in_lin = relayout(input_emb)
    out_lin = relayout(output_emb)

    sc_dots = _make_sc_dots(B, K, E, H)
    pos_dot, neg_dot = sc_dots(
        center.astype(jnp.int32),
        context.astype(jnp.int32),
        negatives.reshape(B * K).astype(jnp.int32),
        in_lin,
        out_lin,
    )

    loss = pl.pallas_call(
        _make_loss_body(B, K),
        out_shape=jax.ShapeDtypeStruct((1, 1), jnp.float32),
        in_specs=[
            pl.BlockSpec(memory_space=pltpu.VMEM),
            pl.BlockSpec(memory_space=pltpu.VMEM),
        ],
        out_specs=pl.BlockSpec(memory_space=pltpu.SMEM),
    )(pos_dot.reshape(B // 128, 128), neg_dot.reshape(B * KPAD // 128, 128))
    return loss.reshape(())


# speedup vs baseline: 2.3101x; 1.0449x over previous
import functools

import jax
import jax.numpy as jnp
from jax import lax
from jax.experimental import pallas as pl
from jax.experimental.pallas import tpu as pltpu
from jax.experimental.pallas import tpu_sc as plsc

LANES = 16
KPAD = 32
BI = 16384


def _make_relayout(V, E):
    NB = pl.cdiv(V, 2 * BI)
    H = NB * BI
    NBV = pl.cdiv(V, BI) - 1

    def body(up_ref, lo_ref, dst_ref):
        t_up = jnp.transpose(up_ref[...])
        t_lo = jnp.transpose(lo_ref[...])
        dst_ref[...] = jnp.concatenate([t_up, t_lo], axis=1)

    call = pl.pallas_call(
        body,
        grid=(NB,),
        in_specs=[
            pl.BlockSpec((E, BI), lambda i: (0, i)),
            pl.BlockSpec((E, BI), lambda i: (0, jnp.minimum(i + NB, NBV))),
        ],
        out_specs=pl.BlockSpec((BI, 2 * E), lambda i: (i, 0)),
        out_shape=jax.ShapeDtypeStruct((H, 2 * E), jnp.float32),
    )

    def relayout(table):
        tT = table.T
        return call(tT, tT).reshape(2 * H, E)

    return relayout, H


def _make_sc_dots(B, K, E, H):
    info = plsc.get_sparse_core_info()
    NW = info.num_cores * info.num_subcores
    rows_per_w = B // NW
    C = 32
    n_pairs = rows_per_w // (2 * C)
    EV = E // LANES
    IDX_BLK = 128

    mesh = plsc.VectorSubcoreMesh(core_axis_name="c", subcore_axis_name="s")

    def one_set():
        return [
            pltpu.VMEM((C,), jnp.int32),
            pltpu.VMEM((C,), jnp.int32),
            pltpu.VMEM((C * K,), jnp.int32),
            pltpu.VMEM((C, E), jnp.float32),
            pltpu.VMEM((C, E), jnp.float32),
            pltpu.VMEM((C * K, E), jnp.float32),
            pltpu.VMEM((C,), jnp.float32),
            pltpu.VMEM((C * KPAD,), jnp.float32),
            pltpu.SemaphoreType.DMA,
        ]

    @functools.partial(
        pl.kernel,
        out_type=[
            jax.ShapeDtypeStruct((B,), jnp.float32),
            jax.ShapeDtypeStruct((B * KPAD,), jnp.float32),
        ],
        mesh=mesh,
        compiler_params=pltpu.CompilerParams(needs_layout_passes=False,
                                             use_tc_tiling_on_sc=False),
        scratch_types=one_set() + one_set(),
    )
    def sc_dots(center_hbm, context_hbm, negflat_hbm, inemb_hbm, outemb_hbm,
                pos_hbm, negdot_hbm, *scratch):
        set0, set1 = scratch[:9], scratch[9:]
        wid = lax.axis_index("s") * info.num_cores + lax.axis_index("c")
        wbase = wid * rows_per_w
        lane = lax.iota(jnp.int32, LANES)

        def remap(ref, n):
            for j in range(n // LANES):
                v = ref[pl.ds(j * LANES, LANES)]
                sel = (v >= H).astype(jnp.int32)
                ref[pl.ds(j * LANES, LANES)] = v * 2 - sel * (2 * H - 1)

        def gather_list(S):
            cidx, oidx, nidx, crow, orow, nrow, _, _, sem = S
            copies = [
                pltpu.make_async_copy(inemb_hbm.at[cidx], crow, sem),
                pltpu.make_async_copy(outemb_hbm.at[oidx], orow, sem),
            ]
            for j in range(C * K // IDX_BLK):
                copies.append(pltpu.make_async_copy(
                    outemb_hbm.at[nidx.at[pl.ds(j * IDX_BLK, IDX_BLK)]],
                    nrow.at[pl.ds(j * IDX_BLK, IDX_BLK)],
                    sem))
            return copies

        def stage_issue(g, S):
            cidx, oidx, nidx = S[0], S[1], S[2]
            sem = S[8]
            base = wbase + g * C
            stages = [
                pltpu.async_copy(center_hbm.at[pl.ds(base, C)], cidx, sem),
                pltpu.async_copy(context_hbm.at[pl.ds(base, C)], oidx, sem),
                pltpu.async_copy(negflat_hbm.at[pl.ds(base * K, C * K)],
                                 nidx, sem),
            ]
            for cp in stages:
                cp.wait()
            remap(cidx, C)
            remap(oidx, C)
            remap(nidx, C * K)
            for cp in gather_list(S):
                cp.start()

        def drain(S):
            for cp in gather_list(S):
                cp.wait()

        def compute_store(g, S):
            _, _, _, crow, orow, nrow, posv, negv, _ = S
            base = wbase + g * C

            def dot_rows(a_ref, a_row, b_ref, b_row):
                acc = (a_ref[a_row, pl.ds(0, LANES)]
                       * b_ref[b_row, pl.ds(0, LANES)])
                for v in range(1, EV):
                    acc = acc + (a_ref[a_row, pl.ds(v * LANES, LANES)]
                                 * b_ref[b_row, pl.ds(v * LANES, LANES)])
                return jnp.sum(acc, axis=0)

            def grp_body(grp, _):
                r0 = grp * LANES
                pvec = jnp.zeros((LANES,), jnp.float32)
                for i in range(LANES):
                    r = r0 + i
                    pvec = jnp.where(lane == i, dot_rows(crow, r, orow, r),
                                     pvec)
                    nvec0 = jnp.zeros((LANES,), jnp.float32)
                    nvec1 = jnp.zeros((LANES,), jnp.float32)
                    for k in range(K):
                        s = dot_rows(crow, r, nrow, r * K + k)
                        if k < LANES:
                            nvec0 = jnp.where(lane == k, s, nvec0)
                        else:
                            nvec1 = jnp.where(lane == (k - LANES), s, nvec1)
                    negv[pl.ds(r * KPAD, LANES)] = nvec0
                    negv[pl.ds(r * KPAD + LANES, LANES)] = nvec1
                posv[pl.ds(r0, LANES)] = pvec
                return 0

            lax.fori_loop(0, C // LANES, grp_body, 0)
            pltpu.sync_copy(posv, pos_hbm.at[pl.ds(base, C)])
            pltpu.sync_copy(negv, negdot_hbm.at[pl.ds(base * KPAD, C * KPAD)])

        stage_issue(0, set0)

        def pair_body(t, _):
            stage_issue(2 * t + 1, set1)
            drain(set0)
            compute_store(2 * t, set0)

            @pl.when(t < n_pairs - 1)
            def _():
                stage_issue(2 * t + 2, set0)

            drain(set1)
            compute_store(2 * t + 1, set1)
            return 0

        lax.fori_loop(0, n_pairs, pair_body, 0)

    return sc_dots


def _make_loss_body(B, K):
    def loss_body(pos_ref, neg_ref, out_ref):
        pos = pos_ref[...]
        neg = neg_ref[...]
        k_of_col = jax.lax.broadcasted_iota(jnp.int32, neg.shape, 1) % KPAD
        pos_term = -jnp.log(jax.nn.sigmoid(pos) + 1e-09)
        neg_term = jnp.where(k_of_col < K,
                             -jnp.log(jax.nn.sigmoid(-neg) + 1e-09), 0.0)
        out_ref[0, 0] = (jnp.sum(pos_term) + jnp.sum(neg_term)) / B
    return loss_body


def kernel(center, context, negatives, input_emb, output_emb):
    B, = center.shape
    K = negatives.shape[1]
    V, E = input_emb.shape

    relayout, H = _make_relayout(V, E)
    in_lin = relayout(input_emb)
    out_lin = relayout(output_emb)

    sc_dots = _make_sc_dots(B, K, E, H)
    pos_dot, neg_dot = sc_dots(
        center.astype(jnp.int32),
        context.astype(jnp.int32),
        negatives.reshape(B * K).astype(jnp.int32),
        in_lin,
        out_lin,
    )

    loss = pl.pallas_call(
        _make_loss_body(B, K),
        out_shape=jax.ShapeDtypeStruct((1, 1), jnp.float32),
        in_specs=[
            pl.BlockSpec(memory_space=pltpu.VMEM),
            pl.BlockSpec(memory_space=pltpu.VMEM),
        ],
        out_specs=pl.BlockSpec(memory_space=pltpu.SMEM),
    )(pos_dot.reshape(B // 128, 128), neg_dot.reshape(B * KPAD // 128, 128))
    return loss.reshape(())
